# baseline (speedup 1.0000x reference)
import functools
import jax
import jax.numpy as jnp
from jax import lax
from jax.experimental import pallas as pl
from jax.experimental.pallas import tpu as pltpu
from jax.experimental.pallas import tpu_sc as plsc

N = 10000
NP = 10240
D = 128
EN = 320000
ED = 160000
CH = 128
NC = 2
NS = 16
NW = NC * NS

TCH_NET = 80
TCH_DAG = 40
BC = 40
ENP = NW * TCH_NET * CH
EDP = NW * TCH_DAG * CH

_PER_SUB = NP // NS


def _sc_deg(ncol2, new2, dcol2, dew2):
    mesh = plsc.VectorSubcoreMesh(core_axis_name="c", subcore_axis_name="s")

    @functools.partial(
        pl.kernel,
        out_type=jax.ShapeDtypeStruct((2 * NC, NP), jnp.float32),
        mesh=mesh,
        scratch_types=[
            pltpu.VMEM((TCH_NET, CH), jnp.int32),
            pltpu.VMEM((TCH_NET * CH,), jnp.float32),
            pltpu.VMEM((_PER_SUB,), jnp.float32),
            pltpu.VMEM_SHARED((NP,), jnp.float32),
            pltpu.VMEM_SHARED((NP,), jnp.float32),
            pltpu.SemaphoreType.DMA,
        ],
    )
    def deg_kernel(ncol_h, new_h, dcol_h, dew_h, out_h,
                   col_v, ew_v, buf_v, deg_net_sh, deg_dag_sh, dsem):
        cid = lax.axis_index("c")
        sid = lax.axis_index("s")
        wid = sid * NC + cid
        zeros16 = jnp.zeros((16,), jnp.float32)

        @pl.loop(0, _PER_SUB // 16)
        def _(i):
            buf_v[pl.ds(i * 16, 16)] = zeros16

        pltpu.sync_copy(buf_v, deg_net_sh.at[pl.ds(sid * _PER_SUB, _PER_SUB)])
        pltpu.sync_copy(buf_v, deg_dag_sh.at[pl.ds(sid * _PER_SUB, _PER_SUB)])
        plsc.subcore_barrier()

        for col_h, ew_h, deg_sh, tch in (
            (ncol_h, new_h, deg_net_sh, TCH_NET),
            (dcol_h, dew_h, deg_dag_sh, TCH_DAG),
        ):
            pltpu.sync_copy(col_h.at[pl.ds(wid * tch, tch)],
                            col_v.at[pl.ds(0, tch)])
            pltpu.sync_copy(ew_h.at[pl.ds(wid * tch * CH, tch * CH)],
                            ew_v.at[pl.ds(0, tch * CH)])

            @pl.loop(0, tch)
            def _(c):
                pltpu.async_copy(ew_v.at[pl.ds(c * CH, CH)],
                                 deg_sh.at[col_v.at[c]], dsem, add=True)

            @pl.loop(0, tch)
            def _(c):
                pltpu.make_async_copy(ew_h.at[pl.ds(0, CH)],
                                      ew_v.at[pl.ds(0, CH)], dsem).wait()

        plsc.subcore_barrier()

        for gi, deg_sh in ((0, deg_net_sh), (1, deg_dag_sh)):
            sl = pl.ds(sid * _PER_SUB, _PER_SUB)
            pltpu.sync_copy(deg_sh.at[sl], out_h.at[2 * gi + cid, sl])

    return deg_kernel(ncol2, new2, dcol2, dew2).reshape(2, NC, NP)


_RB = 1024


def _tc_scale_body(xs_ref, ws_ref, degp_ref, out_ref):
    deg = degp_ref[0, 0] + degp_ref[0, 1] + 1.0
    dinv = jnp.where(deg > 0, lax.rsqrt(deg), 0.0)
    h = jnp.dot(xs_ref[0], ws_ref[0], preferred_element_type=jnp.float32)
    out_ref[0] = dinv[:, None] * h


def _tc_scale(xs, ws, degp):
    return pl.pallas_call(
        _tc_scale_body,
        grid=(2, NP // _RB),
        in_specs=[
            pl.BlockSpec((1, _RB, D), lambda g, i: (g, i, 0)),
            pl.BlockSpec((1, D, D), lambda g, i: (g, 0, 0)),
            pl.BlockSpec((1, NC, _RB), lambda g, i: (g, 0, i)),
        ],
        out_specs=pl.BlockSpec((1, _RB, D), lambda g, i: (g, i, 0)),
        out_shape=jax.ShapeDtypeStruct((2, NP, D), jnp.float32),
    )(xs, ws, degp)


def _sc_agg(gs, nrow2, ncol2, new2, drow2, dcol2, dew2):
    mesh = plsc.VectorSubcoreMesh(core_axis_name="c", subcore_axis_name="s")

    @functools.partial(
        pl.kernel,
        out_type=jax.ShapeDtypeStruct((2, NC, NP, D), jnp.float32),
        mesh=mesh,
        scratch_types=[
            pltpu.VMEM((BC, CH), jnp.int32),
            pltpu.VMEM((BC, CH), jnp.int32),
            pltpu.VMEM((BC * CH,), jnp.float32),
            pltpu.VMEM((CH, D), jnp.float32),
            pltpu.VMEM((CH, D), jnp.float32),
            pltpu.VMEM_SHARED((NP, D), jnp.float32),
            pltpu.SemaphoreType.DMA,
            pltpu.SemaphoreType.DMA,
            pltpu.SemaphoreType.DMA,
            pltpu.SemaphoreType.DMA,
        ],
    )
    def agg_kernel(gs_h, nrow_h, ncol_h, new_h, drow_h, dcol_h, dew_h, out_h,
                   row_v, col_v, ew_v, buf_a, buf_b, acc_sh,
                   gs_a, gs_b, ss_a, ss_b):
        cid = lax.axis_index("c")
        sid = lax.axis_index("s")
        wid = sid * NC + cid
        zeros16 = jnp.zeros((16,), jnp.float32)

        for gi, row_h, col_h, ewe_h, tch in (
            (0, nrow_h, ncol_h, new_h, TCH_NET),
            (1, drow_h, dcol_h, dew_h, TCH_DAG),
        ):
            g_h = gs_h.at[gi]

            @pl.loop(0, CH)
            def _(r):
                rr = buf_a.at[r]
                for d in range(D // 16):
                    rr[pl.ds(d * 16, 16)] = zeros16

            @pl.loop(0, _PER_SUB // CH)
            def _(t):
                base = sid * _PER_SUB + t * CH
                pltpu.sync_copy(buf_a, acc_sh.at[pl.ds(base, CH)])

            plsc.subcore_barrier()

            def g_start(c, buf, sem):
                pltpu.async_copy(g_h.at[row_v.at[c]], buf, sem)

            def g_wait(buf, sem):
                pltpu.make_async_copy(g_h.at[pl.ds(0, CH)], buf, sem).wait()

            def s_start(c, buf, sem):
                pltpu.async_copy(buf, acc_sh.at[col_v.at[c]], sem, add=True)

            def s_wait(buf, sem):
                pltpu.make_async_copy(g_h.at[pl.ds(0, CH)], buf, sem).wait()

            def scale(buf, c):
                @pl.loop(0, CH // 16)
                def _(q):
                    base = q * 16
                    ew16 = ew_v[pl.ds(c * CH + base, 16)]
                    for i in range(16):
                        w = jnp.broadcast_to(ew16[i], (16,))
                        rr = buf.at[base + i]
                        for d in range(D // 16):
                            sl = pl.ds(d * 16, 16)
                            rr[sl] = rr[sl] * w

            @pl.loop(0, tch // BC)
            def _(b):
                cbase = wid * tch + b * BC
                pltpu.sync_copy(row_h.at[pl.ds(cbase, BC)], row_v)
                pltpu.sync_copy(col_h.at[pl.ds(cbase, BC)], col_v)
                pltpu.sync_copy(ewe_h.at[pl.ds(cbase * CH, BC * CH)], ew_v)

                g_start(0, buf_a, gs_a)

                @pl.loop(0, BC, step=2)
                def _(c):
                    g_wait(buf_a, gs_a)

                    @pl.when(c > 0)
                    def _():
                        s_wait(buf_b, ss_b)
                    g_start(c + 1, buf_b, gs_b)
                    scale(buf_a, c)
                    s_start(c, buf_a, ss_a)
                    g_wait(buf_b, gs_b)
                    scale(buf_b, c + 1)
                    s_start(c + 1, buf_b, ss_b)
                    s_wait(buf_a, ss_a)

                    @pl.when(c + 2 < BC)
                    def _():
                        g_start(c + 2, buf_a, gs_a)

                s_wait(buf_b, ss_b)

            plsc.subcore_barrier()

            sl = pl.ds(sid * _PER_SUB, _PER_SUB)
            pltpu.sync_copy(acc_sh.at[sl], out_h.at[gi, cid, sl])

            plsc.subcore_barrier()

    return agg_kernel(gs, nrow2, ncol2, new2, drow2, dcol2, dew2)


_RD = 2048
_NBD = NP // _RD


def _tc_epi_body(accp_ref, gs_ref, degp_ref, bs_ref, w1_ref, b1_ref,
                 w2_ref, b2_ref, out_ref, acc_ref):
    g = pl.program_id(0)
    i = pl.program_id(1)

    deg = degp_ref[0, 0] + degp_ref[0, 1] + 1.0
    dinv = jnp.where(deg > 0, lax.rsqrt(deg), 0.0)
    gmaskf = (lax.broadcasted_iota(jnp.int32, (2, 1), 0) == g).astype(
        jnp.float32)
    b_row = jnp.sum(bs_ref[...] * gmaskf, axis=0)
    p = accp_ref[0, 0] + accp_ref[0, 1] + gs_ref[0]
    y = dinv[:, None] * p + b_row[None, :]
    y = jnp.maximum(y, 0.0)
    rows = i * _RD + lax.broadcasted_iota(jnp.int32, (_RD, 1), 0)
    y = jnp.where(rows < N, y, 0.0)
    nrm = jnp.sqrt(jnp.sum(y * y, axis=1, keepdims=True))
    y = y / jnp.maximum(nrm, 1e-12)
    s = jnp.sum(y, axis=0)

    cur = acc_ref[...]
    gmask = (lax.broadcasted_iota(jnp.int32, (2, 1), 0) == g)
    first = (i == 0)
    base = jnp.where(gmask & first, 0.0, cur)
    accnew = base + jnp.where(gmask, s[None, :], 0.0)
    acc_ref[...] = accnew

    @pl.when((g == 1) & (i == _NBD - 1))
    def _():
        mnet = accnew[0] * (1.0 / N)
        mdag = accnew[1] * (1.0 / N)
        combined = jnp.concatenate([mnet, mdag]).reshape(1, 2 * D)
        h1 = jnp.dot(combined, w1_ref[...],
                     preferred_element_type=jnp.float32) + b1_ref[0][None, :64]
        h1 = jnp.maximum(h1, 0.0)
        val = jnp.sum(h1[0] * w2_ref[0, :64]) + b2_ref[0, 0]
        out_ref[...] = val.reshape(1, 1)


def _tc_epilogue(accp, gs, degp, bs, w1, b1p, w2p, b2p):
    return pl.pallas_call(
        _tc_epi_body,
        grid=(2, _NBD),
        in_specs=[
            pl.BlockSpec((1, NC, _RD, D), lambda g, i: (g, 0, i, 0)),
            pl.BlockSpec((1, _RD, D), lambda g, i: (g, i, 0)),
            pl.BlockSpec((1, NC, _RD), lambda g, i: (g, 0, i)),
            pl.BlockSpec((2, D), lambda g, i: (0, 0)),
            pl.BlockSpec((2 * D, 64), lambda g, i: (0, 0)),
            pl.BlockSpec((1, D), lambda g, i: (0, 0)),
            pl.BlockSpec((1, D), lambda g, i: (0, 0)),
            pl.BlockSpec((1, D), lambda g, i: (0, 0)),
        ],
        out_specs=pl.BlockSpec((1, 1), lambda g, i: (0, 0)),
        out_shape=jax.ShapeDtypeStruct((1, 1), jnp.float32),
        scratch_shapes=[pltpu.VMEM((2, D), jnp.float32)],
    )(accp, gs, degp, bs, w1, b1p, w2p, b2p)


def kernel(net_feat, net_edge_index, net_edge_weights,
           dag_feat, dag_edge_index, dag_edge_weights,
           W_net, b_net, W_dag, b_dag, W1, b1, W2, b2):
    pad = ((0, NP - N), (0, 0))
    xs = jnp.stack([jnp.pad(net_feat, pad), jnp.pad(dag_feat, pad)])
    ws = jnp.stack([W_net, W_dag])
    bs = jnp.stack([b_net, b_dag])

    def _edges(ei, ew, ep):
        e = ew.shape[0]
        fill = (jnp.arange(ep - e, dtype=jnp.int32) % N)
        r = jnp.concatenate([ei[0], fill]).reshape(-1, CH)
        c = jnp.concatenate([ei[1], fill]).reshape(-1, CH)
        w = jnp.pad(ew, (0, ep - e))
        return r, c, w

    nrow2, ncol2, new2 = _edges(net_edge_index, net_edge_weights, ENP)
    drow2, dcol2, dew2 = _edges(dag_edge_index, dag_edge_weights, EDP)

    b1p = jnp.pad(b1, (0, D - 64)).reshape(1, D)
    w2p = jnp.pad(W2.reshape(64), (0, D - 64)).reshape(1, D)
    b2p = jnp.pad(b2, (0, D - 1)).reshape(1, D)

    degp = _sc_deg(ncol2, new2, dcol2, dew2)
    gs = _tc_scale(xs, ws, degp)
    accp = _sc_agg(gs, nrow2, ncol2, new2, drow2, dcol2, dew2)
    out = _tc_epilogue(accp, gs, degp, bs, W1, b1p, w2p, b2p)
    return out.reshape(1)


# speedup vs baseline: 1.1374x; 1.1374x over previous
import functools
import jax
import jax.numpy as jnp
from jax import lax
from jax.experimental import pallas as pl
from jax.experimental.pallas import tpu as pltpu
from jax.experimental.pallas import tpu_sc as plsc

N = 10000
NP = 10240
D = 128
EN = 320000
ED = 160000
CH = 128
NC = 2
NS = 16
NW = NC * NS

TCH_NET = 80
TCH_DAG = 40
BC = 40
ENP = NW * TCH_NET * CH
EDP = NW * TCH_DAG * CH

_PER_SUB = NP // NS


def _sc_deg(nei2, new1, dei2, dew1):
    mesh = plsc.VectorSubcoreMesh(core_axis_name="c", subcore_axis_name="s")

    @functools.partial(
        pl.kernel,
        out_type=jax.ShapeDtypeStruct((2 * NC, NP), jnp.float32),
        mesh=mesh,
        scratch_types=[
            pltpu.VMEM((TCH_NET, CH), jnp.int32),
            pltpu.VMEM((TCH_NET * CH,), jnp.float32),
            pltpu.VMEM((_PER_SUB,), jnp.float32),
            pltpu.VMEM_SHARED((NP,), jnp.float32),
            pltpu.VMEM_SHARED((NP,), jnp.float32),
            pltpu.SemaphoreType.DMA,
        ],
    )
    def deg_kernel(nei_h, new_h, dei_h, dew_h, out_h,
                   col_v, ew_v, buf_v, deg_net_sh, deg_dag_sh, dsem):
        ncol_h = nei_h.at[1]
        dcol_h = dei_h.at[1]
        cid = lax.axis_index("c")
        sid = lax.axis_index("s")
        wid = sid * NC + cid
        zeros16 = jnp.zeros((16,), jnp.float32)

        @pl.loop(0, _PER_SUB // 16)
        def _(i):
            buf_v[pl.ds(i * 16, 16)] = zeros16

        pltpu.sync_copy(buf_v, deg_net_sh.at[pl.ds(sid * _PER_SUB, _PER_SUB)])
        pltpu.sync_copy(buf_v, deg_dag_sh.at[pl.ds(sid * _PER_SUB, _PER_SUB)])
        plsc.subcore_barrier()

        for col_h, ew_h, deg_sh, tch in (
            (ncol_h, new_h, deg_net_sh, TCH_NET),
            (dcol_h, dew_h, deg_dag_sh, TCH_DAG),
        ):
            pltpu.sync_copy(col_h.at[pl.ds(wid * tch, tch)],
                            col_v.at[pl.ds(0, tch)])
            pltpu.sync_copy(ew_h.at[pl.ds(wid * tch * CH, tch * CH)],
                            ew_v.at[pl.ds(0, tch * CH)])

            @pl.loop(0, tch)
            def _(c):
                pltpu.async_copy(ew_v.at[pl.ds(c * CH, CH)],
                                 deg_sh.at[col_v.at[c]], dsem, add=True)

            @pl.loop(0, tch)
            def _(c):
                pltpu.make_async_copy(ew_h.at[pl.ds(0, CH)],
                                      ew_v.at[pl.ds(0, CH)], dsem).wait()

        plsc.subcore_barrier()

        for gi, deg_sh in ((0, deg_net_sh), (1, deg_dag_sh)):
            sl = pl.ds(sid * _PER_SUB, _PER_SUB)
            pltpu.sync_copy(deg_sh.at[sl], out_h.at[2 * gi + cid, sl])

    return deg_kernel(nei2, new1, dei2, dew1).reshape(2, NC, NP)


_RB = 1024


def _tc_scale_body(xs_ref, ws_ref, degp_ref, out_ref):
    deg = degp_ref[0, 0] + degp_ref[0, 1] + 1.0
    dinv = jnp.where(deg > 0, lax.rsqrt(deg), 0.0)
    h = jnp.dot(xs_ref[0], ws_ref[0], preferred_element_type=jnp.float32)
    out_ref[0] = dinv[:, None] * h


def _tc_scale(xs, ws, degp):
    return pl.pallas_call(
        _tc_scale_body,
        grid=(2, NP // _RB),
        in_specs=[
            pl.BlockSpec((1, _RB, D), lambda g, i: (g, i, 0)),
            pl.BlockSpec((1, D, D), lambda g, i: (g, 0, 0)),
            pl.BlockSpec((1, NC, _RB), lambda g, i: (g, 0, i)),
        ],
        out_specs=pl.BlockSpec((1, _RB, D), lambda g, i: (g, i, 0)),
        out_shape=jax.ShapeDtypeStruct((2, NP, D), jnp.float32),
    )(xs, ws, degp)


def _sc_agg(gs, nei2, new1, dei2, dew1):
    mesh = plsc.VectorSubcoreMesh(core_axis_name="c", subcore_axis_name="s")

    @functools.partial(
        pl.kernel,
        out_type=jax.ShapeDtypeStruct((2, NC, NP, D), jnp.float32),
        mesh=mesh,
        scratch_types=[
            pltpu.VMEM((BC, CH), jnp.int32),
            pltpu.VMEM((BC, CH), jnp.int32),
            pltpu.VMEM((BC * CH,), jnp.float32),
            pltpu.VMEM((CH, D), jnp.float32),
            pltpu.VMEM((CH, D), jnp.float32),
            pltpu.VMEM_SHARED((NP, D), jnp.float32),
            pltpu.SemaphoreType.DMA,
            pltpu.SemaphoreType.DMA,
            pltpu.SemaphoreType.DMA,
            pltpu.SemaphoreType.DMA,
        ],
    )
    def agg_kernel(gs_h, nei_h, new_h, dei_h, dew_h, out_h,
                   row_v, col_v, ew_v, buf_a, buf_b, acc_sh,
                   gs_a, gs_b, ss_a, ss_b):
        cid = lax.axis_index("c")
        sid = lax.axis_index("s")
        wid = sid * NC + cid
        zeros16 = jnp.zeros((16,), jnp.float32)

        for gi, row_h, col_h, ewe_h, tch in (
            (0, nei_h.at[0], nei_h.at[1], new_h, TCH_NET),
            (1, dei_h.at[0], dei_h.at[1], dew_h, TCH_DAG),
        ):
            g_h = gs_h.at[gi]

            @pl.loop(0, CH)
            def _(r):
                rr = buf_a.at[r]
                for d in range(D // 16):
                    rr[pl.ds(d * 16, 16)] = zeros16

            @pl.loop(0, _PER_SUB // CH)
            def _(t):
                base = sid * _PER_SUB + t * CH
                pltpu.sync_copy(buf_a, acc_sh.at[pl.ds(base, CH)])

            plsc.subcore_barrier()

            def g_start(c, buf, sem):
                pltpu.async_copy(g_h.at[row_v.at[c]], buf, sem)

            def g_wait(buf, sem):
                pltpu.make_async_copy(g_h.at[pl.ds(0, CH)], buf, sem).wait()

            def s_start(c, buf, sem):
                pltpu.async_copy(buf, acc_sh.at[col_v.at[c]], sem, add=True)

            def s_wait(buf, sem):
                pltpu.make_async_copy(g_h.at[pl.ds(0, CH)], buf, sem).wait()

            def scale(buf, c):
                @pl.loop(0, CH // 16)
                def _(q):
                    base = q * 16
                    ew16 = ew_v[pl.ds(c * CH + base, 16)]
                    for i in range(16):
                        w = jnp.broadcast_to(ew16[i], (16,))
                        rr = buf.at[base + i]
                        for d in range(D // 16):
                            sl = pl.ds(d * 16, 16)
                            rr[sl] = rr[sl] * w

            @pl.loop(0, tch // BC)
            def _(b):
                cbase = wid * tch + b * BC
                pltpu.sync_copy(row_h.at[pl.ds(cbase, BC)], row_v)
                pltpu.sync_copy(col_h.at[pl.ds(cbase, BC)], col_v)
                pltpu.sync_copy(ewe_h.at[pl.ds(cbase * CH, BC * CH)], ew_v)

                g_start(0, buf_a, gs_a)

                @pl.loop(0, BC, step=2)
                def _(c):
                    g_wait(buf_a, gs_a)
                    g_start(c + 1, buf_b, gs_b)
                    scale(buf_a, c)
                    pltpu.sync_copy(buf_a, acc_sh.at[col_v.at[c]], add=True)
                    g_wait(buf_b, gs_b)

                    @pl.when(c + 2 < BC)
                    def _():
                        g_start(c + 2, buf_a, gs_a)
                    scale(buf_b, c + 1)
                    pltpu.sync_copy(buf_b, acc_sh.at[col_v.at[c + 1]],
                                    add=True)

            plsc.subcore_barrier()

            sl = pl.ds(sid * _PER_SUB, _PER_SUB)
            pltpu.sync_copy(acc_sh.at[sl], out_h.at[gi, cid, sl])

            plsc.subcore_barrier()

    return agg_kernel(gs, nei2, new1, dei2, dew1)


_RD = 2048
_NBD = NP // _RD


def _tc_epi_body(accp_ref, gs_ref, degp_ref, bs_ref, w1_ref, b1_ref,
                 w2_ref, b2_ref, out_ref, acc_ref):
    g = pl.program_id(0)
    i = pl.program_id(1)

    deg = degp_ref[0, 0] + degp_ref[0, 1] + 1.0
    dinv = jnp.where(deg > 0, lax.rsqrt(deg), 0.0)
    gmaskf = (lax.broadcasted_iota(jnp.int32, (2, 1), 0) == g).astype(
        jnp.float32)
    b_row = jnp.sum(bs_ref[...] * gmaskf, axis=0)
    p = accp_ref[0, 0] + accp_ref[0, 1] + gs_ref[0]
    y = dinv[:, None] * p + b_row[None, :]
    y = jnp.maximum(y, 0.0)
    rows = i * _RD + lax.broadcasted_iota(jnp.int32, (_RD, 1), 0)
    y = jnp.where(rows < N, y, 0.0)
    nrm = jnp.sqrt(jnp.sum(y * y, axis=1, keepdims=True))
    y = y / jnp.maximum(nrm, 1e-12)
    s = jnp.sum(y, axis=0)

    cur = acc_ref[...]
    gmask = (lax.broadcasted_iota(jnp.int32, (2, 1), 0) == g)
    first = (i == 0)
    base = jnp.where(gmask & first, 0.0, cur)
    accnew = base + jnp.where(gmask, s[None, :], 0.0)
    acc_ref[...] = accnew

    @pl.when((g == 1) & (i == _NBD - 1))
    def _():
        mnet = accnew[0] * (1.0 / N)
        mdag = accnew[1] * (1.0 / N)
        combined = jnp.concatenate([mnet, mdag]).reshape(1, 2 * D)
        h1 = jnp.dot(combined, w1_ref[...],
                     preferred_element_type=jnp.float32) + b1_ref[0][None, :64]
        h1 = jnp.maximum(h1, 0.0)
        val = jnp.sum(h1[0] * w2_ref[0, :64]) + b2_ref[0, 0]
        out_ref[...] = val.reshape(1, 1)


def _tc_epilogue(accp, gs, degp, bs, w1, b1p, w2p, b2p):
    return pl.pallas_call(
        _tc_epi_body,
        grid=(2, _NBD),
        in_specs=[
            pl.BlockSpec((1, NC, _RD, D), lambda g, i: (g, 0, i, 0)),
            pl.BlockSpec((1, _RD, D), lambda g, i: (g, i, 0)),
            pl.BlockSpec((1, NC, _RD), lambda g, i: (g, 0, i)),
            pl.BlockSpec((2, D), lambda g, i: (0, 0)),
            pl.BlockSpec((2 * D, 64), lambda g, i: (0, 0)),
            pl.BlockSpec((1, D), lambda g, i: (0, 0)),
            pl.BlockSpec((1, D), lambda g, i: (0, 0)),
            pl.BlockSpec((1, D), lambda g, i: (0, 0)),
        ],
        out_specs=pl.BlockSpec((1, 1), lambda g, i: (0, 0)),
        out_shape=jax.ShapeDtypeStruct((1, 1), jnp.float32),
        scratch_shapes=[pltpu.VMEM((2, D), jnp.float32)],
    )(accp, gs, degp, bs, w1, b1p, w2p, b2p)


def kernel(net_feat, net_edge_index, net_edge_weights,
           dag_feat, dag_edge_index, dag_edge_weights,
           W_net, b_net, W_dag, b_dag, W1, b1, W2, b2):
    pad = ((0, NP - N), (0, 0))
    xs = jnp.stack([jnp.pad(net_feat, pad), jnp.pad(dag_feat, pad)])
    ws = jnp.stack([W_net, W_dag])
    bs = jnp.stack([b_net, b_dag])

    def _edges(ei, ew, ep):
        e = ew.shape[0]
        fill = (jnp.arange(ep - e, dtype=jnp.int32) % N)
        ei2 = jnp.concatenate(
            [ei, jnp.stack([fill, fill])], axis=1).reshape(2, -1, CH)
        w = jnp.pad(ew, (0, ep - e))
        return ei2, w

    nei2, new1 = _edges(net_edge_index, net_edge_weights, ENP)
    dei2, dew1 = _edges(dag_edge_index, dag_edge_weights, EDP)

    b1p = jnp.pad(b1, (0, D - 64)).reshape(1, D)
    w2p = jnp.pad(W2.reshape(64), (0, D - 64)).reshape(1, D)
    b2p = jnp.pad(b2, (0, D - 1)).reshape(1, D)

    degp = _sc_deg(nei2, new1, dei2, dew1)
    gs = _tc_scale(xs, ws, degp)
    accp = _sc_agg(gs, nei2, new1, dei2, dew1)
    out = _tc_epilogue(accp, gs, degp, bs, W1, b1p, w2p, b2p)
    return out.reshape(1)
